# baseline (speedup 1.0000x reference)
import functools

import jax
import jax.numpy as jnp
from jax import lax
from jax.experimental import pallas as pl
from jax.experimental.pallas import tpu as pltpu
from jax.experimental.pallas import tpu_sc as plsc

N = 10000
E = 320000
NFEAT = 128
NHID = 128
NCLASS = 64
DH = 64

NC = 2
NS = 16
NW = NC * NS
CH = 80
GRP = 5
NBANK = 2
IBANK = 4
GCH = GRP * CH
RPT = 624
REM = N - NS * RPT

_MESH = plsc.VectorSubcoreMesh(core_axis_name="c", subcore_axis_name="s")


def _make_scatter(d, ch):
  ept = E // NW
  nch = ept // ch
  ngrp = nch // GRP
  gch = GRP * ch
  nfull, zrem = RPT // ch, RPT % ch

  @functools.partial(
      pl.kernel,
      out_type=jax.ShapeDtypeStruct((NC, N, d), jnp.float32),
      mesh=_MESH,
      compiler_params=pltpu.CompilerParams(
          use_tc_tiling_on_sc=False, internal_scratch_in_bytes=65536),
      scratch_types=[
          [pltpu.VMEM((gch,), jnp.int32) for _ in range(IBANK)],
          [pltpu.VMEM((gch,), jnp.int32) for _ in range(IBANK)],
          [pltpu.VMEM((ch, d), jnp.float32)
           for _ in range(NBANK * GRP)],
          pltpu.VMEM_SHARED((N, d), jnp.float32),
          pltpu.SemaphoreType.DMA,
          pltpu.SemaphoreType.DMA,
          pltpu.SemaphoreType.DMA,
      ],
  )
  def scatter_kernel(h_hbm, adj_hbm, out_hbm,
                     src_v, dst_v, rows, acc, gsem, ssem, isem):
    c = lax.axis_index("c")
    s = lax.axis_index("s")
    e0 = (c * NS + s) * ept
    gref = h_hbm
    dummy = h_hbm.at[pl.ds(0, ch)]

    def stage_idx(g, bank):
      pltpu.async_copy(
          adj_hbm.at[0, pl.ds(e0 + g * gch, gch)], src_v[bank], isem)
      pltpu.async_copy(
          adj_hbm.at[1, pl.ds(e0 + g * gch, gch)], dst_v[bank], isem)

    def wait_idx():
      for _ in range(2):
        pltpu.make_async_copy(
            adj_hbm.at[0, pl.ds(e0, gch)], src_v[0], isem).wait()

    stage_idx(0, 0)
    stage_idx(1, 1)

    vpr = d // 16

    def zstore(k, _):
      rows[0][lax.div(k, vpr), pl.ds(lax.rem(k, vpr) * 16, 16)] = (
          jnp.zeros((16,), jnp.float32))
      return 0

    lax.fori_loop(0, ch * vpr, zstore, 0)
    r0 = s * RPT
    zcopies = []
    for k in range(nfull):
      zcopies.append(pltpu.async_copy(
          rows[0], acc.at[pl.ds(r0 + k * ch, ch)], gsem))
    zcopies.append(pltpu.async_copy(
        rows[0].at[pl.ds(0, zrem)], acc.at[pl.ds(r0 + nfull * ch, zrem)],
        gsem))

    @pl.when(s == 0)
    def _():
      pltpu.async_copy(
          rows[0].at[pl.ds(0, REM)], acc.at[pl.ds(NS * RPT, REM)],
          gsem).wait()

    for zc in zcopies:
      zc.wait()
    plsc.subcore_barrier()

    def drain(n):
      for _ in range(n):
        pltpu.make_async_copy(dummy, rows[0], ssem).wait()

    def run_group(rbank, ibank):
      gathers = []
      for j in range(GRP):
        gathers.append(pltpu.async_copy(
            gref.at[src_v[ibank].at[pl.ds(j * ch, ch)]],
            rows[rbank * GRP + j], gsem))
      for j in range(GRP):
        gathers[j].wait()
        pltpu.async_copy(
            rows[rbank * GRP + j],
            acc.at[dst_v[ibank].at[pl.ds(j * ch, ch)]],
            ssem, add=True)

    def quad(i, _):
      for t in range(4):
        wait_idx()
        if t < 2:
          @pl.when(i >= 1)
          def _():
            drain(GRP)
        else:
          drain(GRP)
        st = 4 * i + t + 2

        @pl.when(st < ngrp)
        def _():
          stage_idx(st, (t + 2) % 4)

        run_group(t % 2, t)
      return 0

    nquad = ngrp // 4
    lax.fori_loop(0, nquad, quad, 0)
    for t in range(ngrp % 4):
      g = 4 * nquad + t
      wait_idx()
      drain(GRP)
      if g + 2 < ngrp:
        stage_idx(g + 2, (t + 2) % 4)
      run_group(t % 2, t)
    drain(NBANK * GRP)
    plsc.subcore_barrier()
    pltpu.sync_copy(acc.at[pl.ds(r0, RPT)], out_hbm.at[c, pl.ds(r0, RPT)])

    @pl.when(s == 0)
    def _():
      pltpu.sync_copy(acc.at[pl.ds(NS * RPT, REM)],
                      out_hbm.at[c, pl.ds(NS * RPT, REM)])

  return scatter_kernel


_scatter1 = _make_scatter(NHID, 16)


def _mm1_body(x_ref, w_ref, b_ref, o_ref):
  o_ref[...] = (
      jnp.dot(x_ref[...], w_ref[...], preferred_element_type=jnp.float32)
      + b_ref[...])


def _mid_body(p_ref, w_ref, b_ref, o_ref):
  h = p_ref[0] + p_ref[1]
  nrm = jnp.sqrt(jnp.sum(h * h, axis=1, keepdims=True))
  z = h / jnp.maximum(nrm, 1e-12)
  h1 = jnp.maximum(z, 0.0)
  r = (jnp.dot(h1, w_ref[...], preferred_element_type=jnp.float32)
       + b_ref[...])
  o_ref[...] = jnp.concatenate([r, jnp.zeros_like(r)], axis=1)


def _lsm_body(q_ref, o_ref):
  h = (q_ref[0] + q_ref[1])[:, :NCLASS]
  m = jnp.max(h, axis=1, keepdims=True)
  e = jnp.exp(h - m)
  lse = jnp.log(jnp.sum(e, axis=1, keepdims=True))
  o_ref[...] = h - m - lse


def _mm1(x, W0, b0):
  return pl.pallas_call(
      _mm1_body,
      out_shape=jax.ShapeDtypeStruct((N, NHID), jnp.float32),
  )(x, W0, b0)


def _mid(p, W1, b1):
  return pl.pallas_call(
      _mid_body,
      out_shape=jax.ShapeDtypeStruct((N, NHID), jnp.float32),
      compiler_params=pltpu.CompilerParams(
          allow_input_fusion=[True, False, False]),
  )(p, W1, b1)


def _lsm(q):
  return pl.pallas_call(
      _lsm_body,
      out_shape=jax.ShapeDtypeStruct((N, NCLASS), jnp.float32),
      compiler_params=pltpu.CompilerParams(allow_input_fusion=[True]),
  )(q)


def kernel(x, adj, nodes, epoch, W0, b0, W1, b1):
  h = _mm1(x, W0, b0)
  p1 = _scatter1(h, adj)
  h2 = _mid(p1, W1, b1)
  p2 = _scatter1(h2, adj)
  return _lsm(p2)


# speedup vs baseline: 1.4536x; 1.4536x over previous
import functools

import jax
import jax.numpy as jnp
from jax import lax
from jax.experimental import pallas as pl
from jax.experimental.pallas import tpu as pltpu
from jax.experimental.pallas import tpu_sc as plsc

N = 10000
E = 320000
NFEAT = 128
NHID = 128
NCLASS = 64
DH = 64

NC = 2
NS = 16
NW = NC * NS
CH = 80
GRP = 5
NBANK = 2
IBANK = 4
GCH = GRP * CH
RPT = 624
REM = N - NS * RPT

_MESH = plsc.VectorSubcoreMesh(core_axis_name="c", subcore_axis_name="s")


def _make_scatter(d, ch, grp):
  ept = E // NW
  nch = ept // ch
  ech = ept - nch * ch
  ngrp = nch // grp
  assert nch == ngrp * grp and ngrp >= 4
  gch = grp * ch
  nfull, zrem = RPT // ch, RPT % ch

  @functools.partial(
      pl.kernel,
      out_type=jax.ShapeDtypeStruct((NC, N, d), jnp.float32),
      mesh=_MESH,
      compiler_params=pltpu.CompilerParams(use_tc_tiling_on_sc=False),
      scratch_types=[
          [pltpu.VMEM((gch,), jnp.int32) for _ in range(IBANK)],
          [pltpu.VMEM((gch,), jnp.int32) for _ in range(IBANK)],
          [pltpu.VMEM((max(ech, 8),), jnp.int32) for _ in range(2)],
          [pltpu.VMEM((ch, d), jnp.float32)
           for _ in range(NBANK * grp)],
          pltpu.VMEM_SHARED((N, d), jnp.float32),
          pltpu.SemaphoreType.DMA,
          pltpu.SemaphoreType.DMA,
          pltpu.SemaphoreType.DMA,
      ],
  )
  def scatter_kernel(h_hbm, adj_hbm, out_hbm,
                     src_v, dst_v, tidx, rows, acc, gsem, ssem, isem):
    c = lax.axis_index("c")
    s = lax.axis_index("s")
    e0 = (c * NS + s) * ept
    gref = h_hbm
    dummy = h_hbm.at[pl.ds(0, ch)]

    def stage_idx(g, bank):
      pltpu.async_copy(
          adj_hbm.at[0, pl.ds(e0 + g * gch, gch)], src_v[bank], isem)
      pltpu.async_copy(
          adj_hbm.at[1, pl.ds(e0 + g * gch, gch)], dst_v[bank], isem)

    def wait_idx():
      for _ in range(2):
        pltpu.make_async_copy(
            adj_hbm.at[0, pl.ds(e0, gch)], src_v[0], isem).wait()

    stage_idx(0, 0)
    stage_idx(1, 1)

    vpr = d // 16

    def zstore(k, _):
      rows[0][lax.div(k, vpr), pl.ds(lax.rem(k, vpr) * 16, 16)] = (
          jnp.zeros((16,), jnp.float32))
      return 0

    lax.fori_loop(0, ch * vpr, zstore, 0)
    r0 = s * RPT
    zcopies = []
    for k in range(nfull):
      zcopies.append(pltpu.async_copy(
          rows[0], acc.at[pl.ds(r0 + k * ch, ch)], gsem))
    zcopies.append(pltpu.async_copy(
        rows[0].at[pl.ds(0, zrem)], acc.at[pl.ds(r0 + nfull * ch, zrem)],
        gsem))

    @pl.when(s == 0)
    def _():
      pltpu.async_copy(
          rows[0].at[pl.ds(0, REM)], acc.at[pl.ds(NS * RPT, REM)],
          gsem).wait()

    for zc in zcopies:
      zc.wait()
    plsc.subcore_barrier()

    def drain(n):
      for _ in range(n):
        pltpu.make_async_copy(dummy, rows[0], ssem).wait()

    def run_group(rbank, ibank):
      gathers = []
      for j in range(grp):
        gathers.append(pltpu.async_copy(
            gref.at[src_v[ibank].at[pl.ds(j * ch, ch)]],
            rows[rbank * grp + j], gsem))
      for j in range(grp):
        gathers[j].wait()
        pltpu.async_copy(
            rows[rbank * grp + j],
            acc.at[dst_v[ibank].at[pl.ds(j * ch, ch)]],
            ssem, add=True)

    def quad(i, _):
      for t in range(4):
        wait_idx()
        if t < 2:
          @pl.when(i >= 1)
          def _():
            drain(grp)
        else:
          drain(grp)
        st = 4 * i + t + 2

        @pl.when(st < ngrp)
        def _():
          stage_idx(st, (t + 2) % 4)

        run_group(t % 2, t)
      return 0

    nquad = ngrp // 4
    lax.fori_loop(0, nquad, quad, 0)
    for t in range(ngrp % 4):
      g = 4 * nquad + t
      wait_idx()
      drain(grp)
      if g + 2 < ngrp:
        stage_idx(g + 2, (t + 2) % 4)
      run_group(t % 2, t)
    drain(NBANK * grp)
    if ech:
      pltpu.sync_copy(adj_hbm.at[0, pl.ds(e0 + nch * ch, ech)], tidx[0])
      pltpu.sync_copy(adj_hbm.at[1, pl.ds(e0 + nch * ch, ech)], tidx[1])
      pltpu.async_copy(
          gref.at[tidx[0]], rows[0].at[pl.ds(0, ech)], gsem).wait()
      pltpu.sync_copy(rows[0].at[pl.ds(0, ech)], acc.at[tidx[1]], add=True)
    plsc.subcore_barrier()
    pltpu.sync_copy(acc.at[pl.ds(r0, RPT)], out_hbm.at[c, pl.ds(r0, RPT)])

    @pl.when(s == 0)
    def _():
      pltpu.sync_copy(acc.at[pl.ds(NS * RPT, REM)],
                      out_hbm.at[c, pl.ds(NS * RPT, REM)])

  return scatter_kernel


_scatter1 = _make_scatter(NHID, 32, 4)
_scatter2 = _make_scatter(NCLASS, 80, 5)


def _mm1_body(x_ref, w_ref, b_ref, o_ref):
  o_ref[...] = (
      jnp.dot(x_ref[...], w_ref[...], preferred_element_type=jnp.float32)
      + b_ref[...])


def _mid_body(p_ref, w_ref, b_ref, o_ref):
  h = p_ref[0] + p_ref[1]
  nrm = jnp.sqrt(jnp.sum(h * h, axis=1, keepdims=True))
  z = h / jnp.maximum(nrm, 1e-12)
  h1 = jnp.maximum(z, 0.0)
  o_ref[...] = (
      jnp.dot(h1, w_ref[...], preferred_element_type=jnp.float32)
      + b_ref[...])


def _lsm_body(q_ref, o_ref):
  h = q_ref[0] + q_ref[1]
  m = jnp.max(h, axis=1, keepdims=True)
  e = jnp.exp(h - m)
  lse = jnp.log(jnp.sum(e, axis=1, keepdims=True))
  o_ref[...] = h - m - lse


def _mm1(x, W0, b0):
  return pl.pallas_call(
      _mm1_body,
      out_shape=jax.ShapeDtypeStruct((N, NHID), jnp.float32),
  )(x, W0, b0)


def _mid(p, W1, b1):
  return pl.pallas_call(
      _mid_body,
      out_shape=jax.ShapeDtypeStruct((N, NCLASS), jnp.float32),
      compiler_params=pltpu.CompilerParams(
          allow_input_fusion=[True, False, False]),
  )(p, W1, b1)


def _lsm(q):
  return pl.pallas_call(
      _lsm_body,
      out_shape=jax.ShapeDtypeStruct((N, NCLASS), jnp.float32),
      compiler_params=pltpu.CompilerParams(allow_input_fusion=[True]),
  )(q)


def kernel(x, adj, nodes, epoch, W0, b0, W1, b1):
  h = _mm1(x, W0, b0)
  p1 = _scatter1(h, adj)
  h2 = _mid(p1, W1, b1)
  p2 = _scatter2(h2, adj)
  return _lsm(p2)


# speedup vs baseline: 1.5260x; 1.0498x over previous
import functools

import jax
import jax.numpy as jnp
from jax import lax
from jax.experimental import pallas as pl
from jax.experimental.pallas import tpu as pltpu
from jax.experimental.pallas import tpu_sc as plsc

N = 10000
E = 320000
NFEAT = 128
NHID = 128
NCLASS = 64
DH = 64

NC = 2
NS = 16
NW = NC * NS
CH = 80
GRP = 5
NBANK = 2
IBANK = 4
GCH = GRP * CH
RPT = 624
REM = N - NS * RPT

_MESH = plsc.VectorSubcoreMesh(core_axis_name="c", subcore_axis_name="s")


def _make_scatter(d, ch, grp):
  ept = E // NW
  nch = ept // ch
  ech = ept - nch * ch
  ngrp = nch // grp
  assert nch == ngrp * grp and ngrp >= 4
  gch = grp * ch
  nfull, zrem = RPT // ch, RPT % ch

  @functools.partial(
      pl.kernel,
      out_type=jax.ShapeDtypeStruct((NC, N, d), jnp.float32),
      mesh=_MESH,
      compiler_params=pltpu.CompilerParams(use_tc_tiling_on_sc=False),
      scratch_types=[
          [pltpu.VMEM((gch,), jnp.int32) for _ in range(IBANK)],
          [pltpu.VMEM((gch,), jnp.int32) for _ in range(IBANK)],
          [pltpu.VMEM((max(ech, 8),), jnp.int32) for _ in range(2)],
          [pltpu.VMEM((ch, d), jnp.float32)
           for _ in range(NBANK * grp)],
          pltpu.VMEM_SHARED((N, d), jnp.float32),
          pltpu.SemaphoreType.DMA,
          pltpu.SemaphoreType.DMA,
          pltpu.SemaphoreType.DMA,
      ],
  )
  def scatter_kernel(h_hbm, adj_hbm, out_hbm,
                     src_v, dst_v, tidx, rows, acc, gsem, ssem, isem):
    c = lax.axis_index("c")
    s = lax.axis_index("s")
    e0 = (c * NS + s) * ept
    gref = h_hbm
    dummy = h_hbm.at[pl.ds(0, ch)]

    def stage_idx(g, bank):
      pltpu.async_copy(
          adj_hbm.at[0, pl.ds(e0 + g * gch, gch)], src_v[bank], isem)
      pltpu.async_copy(
          adj_hbm.at[1, pl.ds(e0 + g * gch, gch)], dst_v[bank], isem)

    def wait_idx():
      for _ in range(2):
        pltpu.make_async_copy(
            adj_hbm.at[0, pl.ds(e0, gch)], src_v[0], isem).wait()

    stage_idx(0, 0)
    stage_idx(1, 1)

    vpr = d // 16

    def zstore(k, _):
      rows[0][lax.div(k, vpr), pl.ds(lax.rem(k, vpr) * 16, 16)] = (
          jnp.zeros((16,), jnp.float32))
      return 0

    lax.fori_loop(0, ch * vpr, zstore, 0)
    r0 = s * RPT
    zcopies = []
    for k in range(nfull):
      zcopies.append(pltpu.async_copy(
          rows[0], acc.at[pl.ds(r0 + k * ch, ch)], gsem))
    zcopies.append(pltpu.async_copy(
        rows[0].at[pl.ds(0, zrem)], acc.at[pl.ds(r0 + nfull * ch, zrem)],
        gsem))

    @pl.when(s == 0)
    def _():
      pltpu.async_copy(
          rows[0].at[pl.ds(0, REM)], acc.at[pl.ds(NS * RPT, REM)],
          gsem).wait()

    for zc in zcopies:
      zc.wait()
    plsc.subcore_barrier()

    def drain(n):
      for _ in range(n):
        pltpu.make_async_copy(dummy, rows[0], ssem).wait()

    def run_group(rbank, ibank):
      gathers = []
      for j in range(grp):
        gathers.append(pltpu.async_copy(
            gref.at[src_v[ibank].at[pl.ds(j * ch, ch)]],
            rows[rbank * grp + j], gsem))
      for j in range(grp):
        gathers[j].wait()
        pltpu.async_copy(
            rows[rbank * grp + j],
            acc.at[dst_v[ibank].at[pl.ds(j * ch, ch)]],
            ssem, add=True)

    def quad(i, _):
      for t in range(4):
        wait_idx()
        if t < 2:
          @pl.when(i >= 1)
          def _():
            drain(grp)
        else:
          drain(grp)
        st = 4 * i + t + 2

        @pl.when(st < ngrp)
        def _():
          stage_idx(st, (t + 2) % 4)

        run_group(t % 2, t)
      return 0

    nquad = ngrp // 4
    lax.fori_loop(0, nquad, quad, 0)
    for t in range(ngrp % 4):
      g = 4 * nquad + t
      wait_idx()
      drain(grp)
      if g + 2 < ngrp:
        stage_idx(g + 2, (t + 2) % 4)
      run_group(t % 2, t)
    drain(NBANK * grp)
    if ech:
      pltpu.sync_copy(adj_hbm.at[0, pl.ds(e0 + nch * ch, ech)], tidx[0])
      pltpu.sync_copy(adj_hbm.at[1, pl.ds(e0 + nch * ch, ech)], tidx[1])
      pltpu.async_copy(
          gref.at[tidx[0]], rows[0].at[pl.ds(0, ech)], gsem).wait()
      pltpu.sync_copy(rows[0].at[pl.ds(0, ech)], acc.at[tidx[1]], add=True)
    plsc.subcore_barrier()
    pltpu.sync_copy(acc.at[pl.ds(r0, RPT)], out_hbm.at[c, pl.ds(r0, RPT)])

    @pl.when(s == 0)
    def _():
      pltpu.sync_copy(acc.at[pl.ds(NS * RPT, REM)],
                      out_hbm.at[c, pl.ds(NS * RPT, REM)])

  return scatter_kernel


_scatter1 = _make_scatter(NHID, 32, 6)
_scatter2 = _make_scatter(NCLASS, 40, 10)


def _mm1_body(x_ref, w_ref, b_ref, o_ref):
  o_ref[...] = (
      jnp.dot(x_ref[...], w_ref[...], preferred_element_type=jnp.float32)
      + b_ref[...])


def _mid_body(p_ref, w_ref, b_ref, o_ref):
  h = p_ref[0] + p_ref[1]
  nrm = jnp.sqrt(jnp.sum(h * h, axis=1, keepdims=True))
  z = h / jnp.maximum(nrm, 1e-12)
  h1 = jnp.maximum(z, 0.0)
  o_ref[...] = (
      jnp.dot(h1, w_ref[...], preferred_element_type=jnp.float32)
      + b_ref[...])


def _lsm_body(q_ref, o_ref):
  h = q_ref[0] + q_ref[1]
  m = jnp.max(h, axis=1, keepdims=True)
  e = jnp.exp(h - m)
  lse = jnp.log(jnp.sum(e, axis=1, keepdims=True))
  o_ref[...] = h - m - lse


def _mm1(x, W0, b0):
  return pl.pallas_call(
      _mm1_body,
      out_shape=jax.ShapeDtypeStruct((N, NHID), jnp.float32),
  )(x, W0, b0)


def _mid(p, W1, b1):
  return pl.pallas_call(
      _mid_body,
      out_shape=jax.ShapeDtypeStruct((N, NCLASS), jnp.float32),
      compiler_params=pltpu.CompilerParams(
          allow_input_fusion=[True, False, False]),
  )(p, W1, b1)


def _lsm(q):
  return pl.pallas_call(
      _lsm_body,
      out_shape=jax.ShapeDtypeStruct((N, NCLASS), jnp.float32),
      compiler_params=pltpu.CompilerParams(allow_input_fusion=[True]),
  )(q)


def kernel(x, adj, nodes, epoch, W0, b0, W1, b1):
  h = _mm1(x, W0, b0)
  p1 = _scatter1(h, adj)
  h2 = _mid(p1, W1, b1)
  p2 = _scatter2(h2, adj)
  return _lsm(p2)


# speedup vs baseline: 1.5871x; 1.0400x over previous
import functools

import jax
import jax.numpy as jnp
from jax import lax
from jax.experimental import pallas as pl
from jax.experimental.pallas import tpu as pltpu
from jax.experimental.pallas import tpu_sc as plsc

N = 10000
E = 320000
NFEAT = 128
NHID = 128
NCLASS = 64
DH = 64

NC = 2
NS = 16
NW = NC * NS
CH = 80
GRP = 5
NBANK = 2
IBANK = 4
GCH = GRP * CH
RPT = 624
REM = N - NS * RPT

_MESH = plsc.VectorSubcoreMesh(core_axis_name="c", subcore_axis_name="s")


def _make_scatter(d, ch, grp, interleave=False):
  ept = E // NW
  nch = ept // ch
  ech = ept - nch * ch
  ngrp = nch // grp
  assert nch == ngrp * grp and ngrp >= 4
  gch = grp * ch
  nfull, zrem = RPT // ch, RPT % ch

  @functools.partial(
      pl.kernel,
      out_type=jax.ShapeDtypeStruct(
          (N, NC * d) if interleave else (NC, N, d), jnp.float32),
      mesh=_MESH,
      compiler_params=pltpu.CompilerParams(use_tc_tiling_on_sc=False),
      scratch_types=[
          [pltpu.VMEM((gch,), jnp.int32) for _ in range(IBANK)],
          [pltpu.VMEM((gch,), jnp.int32) for _ in range(IBANK)],
          [pltpu.VMEM((max(ech, 8),), jnp.int32) for _ in range(2)],
          [pltpu.VMEM((ch, d), jnp.float32)
           for _ in range(NBANK * grp)],
          pltpu.VMEM_SHARED((N, d), jnp.float32),
          pltpu.SemaphoreType.DMA,
          pltpu.SemaphoreType.DMA,
          pltpu.SemaphoreType.DMA,
      ],
  )
  def scatter_kernel(h_hbm, adj_hbm, out_hbm,
                     src_v, dst_v, tidx, rows, acc, gsem, ssem, isem):
    c = lax.axis_index("c")
    s = lax.axis_index("s")
    e0 = (c * NS + s) * ept
    gref = h_hbm
    dummy = h_hbm.at[pl.ds(0, ch)]

    def stage_idx(g, bank):
      pltpu.async_copy(
          adj_hbm.at[0, pl.ds(e0 + g * gch, gch)], src_v[bank], isem)
      pltpu.async_copy(
          adj_hbm.at[1, pl.ds(e0 + g * gch, gch)], dst_v[bank], isem)

    def wait_idx():
      for _ in range(2):
        pltpu.make_async_copy(
            adj_hbm.at[0, pl.ds(e0, gch)], src_v[0], isem).wait()

    stage_idx(0, 0)
    stage_idx(1, 1)

    vpr = d // 16

    def zstore(k, _):
      rows[0][lax.div(k, vpr), pl.ds(lax.rem(k, vpr) * 16, 16)] = (
          jnp.zeros((16,), jnp.float32))
      return 0

    lax.fori_loop(0, ch * vpr, zstore, 0)
    r0 = s * RPT
    zcopies = []
    for k in range(nfull):
      zcopies.append(pltpu.async_copy(
          rows[0], acc.at[pl.ds(r0 + k * ch, ch)], gsem))
    zcopies.append(pltpu.async_copy(
        rows[0].at[pl.ds(0, zrem)], acc.at[pl.ds(r0 + nfull * ch, zrem)],
        gsem))

    @pl.when(s == 0)
    def _():
      pltpu.async_copy(
          rows[0].at[pl.ds(0, REM)], acc.at[pl.ds(NS * RPT, REM)],
          gsem).wait()

    for zc in zcopies:
      zc.wait()
    plsc.subcore_barrier()

    def drain(n):
      for _ in range(n):
        pltpu.make_async_copy(dummy, rows[0], ssem).wait()

    def run_group(rbank, ibank):
      gathers = []
      for j in range(grp):
        gathers.append(pltpu.async_copy(
            gref.at[src_v[ibank].at[pl.ds(j * ch, ch)]],
            rows[rbank * grp + j], gsem))
      for j in range(grp):
        gathers[j].wait()
        pltpu.async_copy(
            rows[rbank * grp + j],
            acc.at[dst_v[ibank].at[pl.ds(j * ch, ch)]],
            ssem, add=True)

    def quad(i, _):
      for t in range(4):
        wait_idx()
        if t < 2:
          @pl.when(i >= 1)
          def _():
            drain(grp)
        else:
          drain(grp)
        st = 4 * i + t + 2

        @pl.when(st < ngrp)
        def _():
          stage_idx(st, (t + 2) % 4)

        run_group(t % 2, t)
      return 0

    nquad = ngrp // 4
    lax.fori_loop(0, nquad, quad, 0)
    for t in range(ngrp % 4):
      g = 4 * nquad + t
      wait_idx()
      drain(grp)
      if g + 2 < ngrp:
        stage_idx(g + 2, (t + 2) % 4)
      run_group(t % 2, t)
    drain(NBANK * grp)
    if ech:
      pltpu.sync_copy(adj_hbm.at[0, pl.ds(e0 + nch * ch, ech)], tidx[0])
      pltpu.sync_copy(adj_hbm.at[1, pl.ds(e0 + nch * ch, ech)], tidx[1])
      pltpu.async_copy(
          gref.at[tidx[0]], rows[0].at[pl.ds(0, ech)], gsem).wait()
      pltpu.sync_copy(rows[0].at[pl.ds(0, ech)], acc.at[tidx[1]], add=True)
    plsc.subcore_barrier()
    if interleave:
      o_main = out_hbm.at[pl.ds(r0, RPT), pl.ds(c * d, d)]
      o_rem = out_hbm.at[pl.ds(NS * RPT, REM), pl.ds(c * d, d)]
    else:
      o_main = out_hbm.at[c, pl.ds(r0, RPT)]
      o_rem = out_hbm.at[c, pl.ds(NS * RPT, REM)]
    pltpu.sync_copy(acc.at[pl.ds(r0, RPT)], o_main)

    @pl.when(s == 0)
    def _():
      pltpu.sync_copy(acc.at[pl.ds(NS * RPT, REM)], o_rem)

  return scatter_kernel


_scatter1 = _make_scatter(NHID, 32, 6)
_scatter2 = _make_scatter(NCLASS, 40, 10, interleave=True)


def _mm1_body(x_ref, w_ref, b_ref, o_ref):
  o_ref[...] = (
      jnp.dot(x_ref[...], w_ref[...], preferred_element_type=jnp.float32)
      + b_ref[...])


def _mid_body(p_ref, w_ref, b_ref, o_ref):
  h = p_ref[0] + p_ref[1]
  nrm = jnp.sqrt(jnp.sum(h * h, axis=1, keepdims=True))
  z = h / jnp.maximum(nrm, 1e-12)
  h1 = jnp.maximum(z, 0.0)
  o_ref[...] = (
      jnp.dot(h1, w_ref[...], preferred_element_type=jnp.float32)
      + b_ref[...])


def _lsm_body(q_ref, o_ref):
  h = q_ref[:, :NCLASS] + q_ref[:, NCLASS:]
  m = jnp.max(h, axis=1, keepdims=True)
  e = jnp.exp(h - m)
  lse = jnp.log(jnp.sum(e, axis=1, keepdims=True))
  o_ref[...] = h - m - lse


def _mm1(x, W0, b0):
  return pl.pallas_call(
      _mm1_body,
      out_shape=jax.ShapeDtypeStruct((N, NHID), jnp.float32),
  )(x, W0, b0)


def _mid(p, W1, b1):
  return pl.pallas_call(
      _mid_body,
      out_shape=jax.ShapeDtypeStruct((N, NCLASS), jnp.float32),
      compiler_params=pltpu.CompilerParams(
          allow_input_fusion=[True, False, False]),
  )(p, W1, b1)


def _lsm(q):
  return pl.pallas_call(
      _lsm_body,
      out_shape=jax.ShapeDtypeStruct((N, NCLASS), jnp.float32),
      compiler_params=pltpu.CompilerParams(allow_input_fusion=[True]),
  )(q)


def kernel(x, adj, nodes, epoch, W0, b0, W1, b1):
  h = _mm1(x, W0, b0)
  p1 = _scatter1(h, adj)
  h2 = _mid(p1, W1, b1)
  p2 = _scatter2(h2, adj)
  return _lsm(p2)


# speedup vs baseline: 1.5880x; 1.0006x over previous
import functools

import jax
import jax.numpy as jnp
from jax import lax
from jax.experimental import pallas as pl
from jax.experimental.pallas import tpu as pltpu
from jax.experimental.pallas import tpu_sc as plsc

N = 10000
E = 320000
NFEAT = 128
NHID = 128
NCLASS = 64
DH = 64

NC = 2
NS = 16
NW = NC * NS
CH = 80
GRP = 5
NBANK = 2
IBANK = 4
GCH = GRP * CH
RPT = 624
REM = N - NS * RPT

_MESH = plsc.VectorSubcoreMesh(core_axis_name="c", subcore_axis_name="s")


def _make_scatter(d, ch, grp, interleave=False):
  ept = E // NW
  nch = ept // ch
  ech = ept - nch * ch
  ngrp = nch // grp
  assert nch == ngrp * grp and ngrp >= 4
  gch = grp * ch
  nfull, zrem = RPT // ch, RPT % ch

  @functools.partial(
      pl.kernel,
      out_type=jax.ShapeDtypeStruct(
          (N, NC * d) if interleave else (NC, N, d), jnp.float32),
      mesh=_MESH,
      compiler_params=pltpu.CompilerParams(use_tc_tiling_on_sc=False),
      scratch_types=[
          [pltpu.VMEM((gch,), jnp.int32) for _ in range(IBANK)],
          [pltpu.VMEM((gch,), jnp.int32) for _ in range(IBANK)],
          [pltpu.VMEM((max(ech, 8),), jnp.int32) for _ in range(2)],
          [pltpu.VMEM((ch, d), jnp.float32)
           for _ in range(NBANK * grp)],
          pltpu.VMEM_SHARED((N, d), jnp.float32),
          pltpu.SemaphoreType.DMA,
          pltpu.SemaphoreType.DMA,
          pltpu.SemaphoreType.DMA,
      ],
  )
  def scatter_kernel(h_hbm, adj_hbm, out_hbm,
                     src_v, dst_v, tidx, rows, acc, gsem, ssem, isem):
    c = lax.axis_index("c")
    s = lax.axis_index("s")
    e0 = (c * NS + s) * ept
    gref = h_hbm
    dummy = h_hbm.at[pl.ds(0, ch)]

    def stage_idx(g, bank):
      pltpu.async_copy(
          adj_hbm.at[0, pl.ds(e0 + g * gch, gch)], src_v[bank], isem)
      pltpu.async_copy(
          adj_hbm.at[1, pl.ds(e0 + g * gch, gch)], dst_v[bank], isem)

    def wait_idx():
      for _ in range(2):
        pltpu.make_async_copy(
            adj_hbm.at[0, pl.ds(e0, gch)], src_v[0], isem).wait()

    stage_idx(0, 0)
    stage_idx(1, 1)

    vpr = d // 16

    def zstore(k, _):
      rows[0][lax.div(k, vpr), pl.ds(lax.rem(k, vpr) * 16, 16)] = (
          jnp.zeros((16,), jnp.float32))
      return 0

    lax.fori_loop(0, ch * vpr, zstore, 0)
    r0 = s * RPT
    zcopies = []
    for k in range(nfull):
      zcopies.append(pltpu.async_copy(
          rows[0], acc.at[pl.ds(r0 + k * ch, ch)], gsem))
    zcopies.append(pltpu.async_copy(
        rows[0].at[pl.ds(0, zrem)], acc.at[pl.ds(r0 + nfull * ch, zrem)],
        gsem))

    @pl.when(s == 0)
    def _():
      pltpu.async_copy(
          rows[0].at[pl.ds(0, REM)], acc.at[pl.ds(NS * RPT, REM)],
          gsem).wait()

    for zc in zcopies:
      zc.wait()
    plsc.subcore_barrier()

    def drain(n):
      for _ in range(n):
        pltpu.make_async_copy(dummy, rows[0], ssem).wait()

    def run_group(rbank, ibank):
      gathers = []
      for j in range(grp):
        gathers.append(pltpu.async_copy(
            gref.at[src_v[ibank].at[pl.ds(j * ch, ch)]],
            rows[rbank * grp + j], gsem))
      for j in range(grp):
        gathers[j].wait()
        pltpu.async_copy(
            rows[rbank * grp + j],
            acc.at[dst_v[ibank].at[pl.ds(j * ch, ch)]],
            ssem, add=True)

    def quad(i, _):
      for t in range(4):
        wait_idx()
        if t < 2:
          @pl.when(i >= 1)
          def _():
            drain(grp)
        else:
          drain(grp)
        st = 4 * i + t + 2

        @pl.when(st < ngrp)
        def _():
          stage_idx(st, (t + 2) % 4)

        run_group(t % 2, t)
      return 0

    nquad = ngrp // 4
    lax.fori_loop(0, nquad, quad, 0)
    for t in range(ngrp % 4):
      g = 4 * nquad + t
      wait_idx()
      drain(grp)
      if g + 2 < ngrp:
        stage_idx(g + 2, (t + 2) % 4)
      run_group(t % 2, t)
    drain(NBANK * grp)
    if ech:
      pltpu.sync_copy(adj_hbm.at[0, pl.ds(e0 + nch * ch, ech)], tidx[0])
      pltpu.sync_copy(adj_hbm.at[1, pl.ds(e0 + nch * ch, ech)], tidx[1])
      pltpu.async_copy(
          gref.at[tidx[0]], rows[0].at[pl.ds(0, ech)], gsem).wait()
      pltpu.sync_copy(rows[0].at[pl.ds(0, ech)], acc.at[tidx[1]], add=True)
    plsc.subcore_barrier()
    if interleave:
      o_main = out_hbm.at[pl.ds(r0, RPT), pl.ds(c * d, d)]
      o_rem = out_hbm.at[pl.ds(NS * RPT, REM), pl.ds(c * d, d)]
    else:
      o_main = out_hbm.at[c, pl.ds(r0, RPT)]
      o_rem = out_hbm.at[c, pl.ds(NS * RPT, REM)]
    pltpu.sync_copy(acc.at[pl.ds(r0, RPT)], o_main)

    @pl.when(s == 0)
    def _():
      pltpu.sync_copy(acc.at[pl.ds(NS * RPT, REM)], o_rem)

  return scatter_kernel


_scatter1 = _make_scatter(NHID, 24, 8)
_scatter2 = _make_scatter(NCLASS, 40, 10, interleave=True)


def _mm1_body(x_ref, w_ref, b_ref, o_ref):
  o_ref[...] = (
      jnp.dot(x_ref[...], w_ref[...], preferred_element_type=jnp.float32)
      + b_ref[...])


def _mid_body(p_ref, w_ref, b_ref, o_ref):
  h = p_ref[0] + p_ref[1]
  nrm = jnp.sqrt(jnp.sum(h * h, axis=1, keepdims=True))
  z = h / jnp.maximum(nrm, 1e-12)
  h1 = jnp.maximum(z, 0.0)
  o_ref[...] = (
      jnp.dot(h1, w_ref[...], preferred_element_type=jnp.float32)
      + b_ref[...])


def _lsm_body(q_ref, o_ref):
  h = q_ref[:, :NCLASS] + q_ref[:, NCLASS:]
  m = jnp.max(h, axis=1, keepdims=True)
  e = jnp.exp(h - m)
  lse = jnp.log(jnp.sum(e, axis=1, keepdims=True))
  o_ref[...] = h - m - lse


def _mm1(x, W0, b0):
  return pl.pallas_call(
      _mm1_body,
      out_shape=jax.ShapeDtypeStruct((N, NHID), jnp.float32),
  )(x, W0, b0)


def _mid(p, W1, b1):
  return pl.pallas_call(
      _mid_body,
      out_shape=jax.ShapeDtypeStruct((N, NCLASS), jnp.float32),
      compiler_params=pltpu.CompilerParams(
          allow_input_fusion=[True, False, False]),
  )(p, W1, b1)


def _lsm(q):
  return pl.pallas_call(
      _lsm_body,
      out_shape=jax.ShapeDtypeStruct((N, NCLASS), jnp.float32),
      compiler_params=pltpu.CompilerParams(allow_input_fusion=[True]),
  )(q)


def kernel(x, adj, nodes, epoch, W0, b0, W1, b1):
  h = _mm1(x, W0, b0)
  p1 = _scatter1(h, adj)
  h2 = _mid(p1, W1, b1)
  p2 = _scatter2(h2, adj)
  return _lsm(p2)
